# baseline (speedup 1.0000x reference)
import functools

import jax
import jax.numpy as jnp
from jax import lax
from jax.experimental import pallas as pl
from jax.experimental.pallas import tpu as pltpu
from jax.experimental.pallas import tpu_sc as plsc

L = 16


def _take(v, idx):
    return lax.gather(
        v, idx[:, None],
        dimension_numbers=lax.GatherDimensionNumbers(
            offset_dims=(), collapsed_slice_dims=(0,), start_index_map=(0,)),
        slice_sizes=(1,),
        mode=lax.GatherScatterMode.PROMISE_IN_BOUNDS)


def _bilerp_kernel(n1, n2, ne, b, nw, q1_hbm, q2_hbm, table_hbm, out_hbm,
                   q1_v, q2_v, idx_v, rbuf0, rbuf1, obuf0, obuf1,
                   sg0, sg1, so0, so1):
    nq = b // nw
    n_chunks = nq // 2
    wid = lax.axis_index("s") * 2 + lax.axis_index("c")
    qbase = wid * nq

    pltpu.sync_copy(q1_hbm.at[pl.ds(qbase, nq)], q1_v)
    pltpu.sync_copy(q2_hbm.at[pl.ds(qbase, nq)], q2_v)

    iota = lax.iota(jnp.int32, L)
    k = iota & 3
    pattern = (k & 1) + jnp.int32(n2) * (k >> 1)

    def cell(qvec, n):
        t = qvec * jnp.float32(n - 1)
        i = jnp.minimum(t.astype(jnp.int32), jnp.int32(n - 2))
        w = t - i.astype(jnp.float32)
        return i, w

    def group_vals(g):
        q1g = q1_v[pl.ds(g * L, L)]
        q2g = q2_v[pl.ds(g * L, L)]
        i1, w1 = cell(q1g, n1)
        i2, w2 = cell(q2g, n2)
        r00 = i1 * jnp.int32(n2) + i2
        return r00, w1, w2

    @plsc.parallel_loop(0, nq // 4, unroll=4)
    def build_idx(p):
        r00, _w1, _w2 = group_vals(p >> 2)
        sel = (4 * p) & (L - 1)
        idx_v[pl.ds(p * L, L)] = _take(r00, sel + (iota >> 2)) + pattern

    def fire_gather(c, rbuf, sem):
        pltpu.async_copy(table_hbm.at[idx_v.at[pl.ds(8 * c, 8)]], rbuf, sem)

    def wait_gather(rbuf, sem):
        pltpu.make_async_copy(
            table_hbm.at[idx_v.at[pl.ds(0, 8)]], rbuf, sem).wait()

    def compute(c, rbuf, obuf):
        r00, w1g, w2g = group_vals(c >> 3)
        lane = (2 * c) & (L - 1)
        one = jnp.float32(1.0)
        ws = []
        for jj in range(2):
            w1 = _take(w1g, jnp.full((L,), lane + jj, jnp.int32))
            w2 = _take(w2g, jnp.full((L,), lane + jj, jnp.int32))
            w11 = w1 * w2
            w10 = w1 - w11
            w01 = w2 - w11
            w00 = (one - w1) - w01
            ws.append((w00, w01, w10, w11))

        @plsc.parallel_loop(0, ne // L, unroll=8)
        def body(e):
            sl = pl.ds(e * L, L)
            for jj in range(2):
                w00, w01, w10, w11 = ws[jj]
                s00 = rbuf[4 * jj + 0, sl]
                s01 = rbuf[4 * jj + 1, sl]
                s10 = rbuf[4 * jj + 2, sl]
                s11 = rbuf[4 * jj + 3, sl]
                obuf[jj, sl] = \
                    (w00 * s00 + w01 * s01) + (w10 * s10 + w11 * s11)

    def fire_out(c, obuf, sem):
        pltpu.async_copy(obuf, out_hbm.at[pl.ds(qbase + 2 * c, 2)], sem)

    def wait_out(obuf, sem):
        pltpu.make_async_copy(obuf, out_hbm.at[pl.ds(0, 2)], sem).wait()

    fire_gather(0, rbuf0, sg0)
    fire_gather(1, rbuf1, sg1)

    def step(i, _):
        wait_gather(rbuf0, sg0)

        @pl.when(i > 0)
        def _():
            wait_out(obuf0, so0)

        compute(2 * i, rbuf0, obuf0)
        fire_out(2 * i, obuf0, so0)

        @pl.when(2 * i + 2 < n_chunks)
        def _():
            fire_gather(2 * i + 2, rbuf0, sg0)

        wait_gather(rbuf1, sg1)

        @pl.when(i > 0)
        def _():
            wait_out(obuf1, so1)

        compute(2 * i + 1, rbuf1, obuf1)
        fire_out(2 * i + 1, obuf1, so1)

        @pl.when(2 * i + 3 < n_chunks)
        def _():
            fire_gather(2 * i + 3, rbuf1, sg1)

        return 0

    lax.fori_loop(0, n_chunks // 2, step, 0, unroll=False)

    wait_out(obuf0, so0)
    wait_out(obuf1, so1)


def _tc_lerp_body(n1, n2, r00_ref, q1_ref, q2_ref,
                  s00_ref, s01_ref, s10_ref, s11_ref, out_ref):
    bq = pl.program_id(0)
    r = r00_ref[bq]
    i1 = r // jnp.int32(n2)
    i2 = r - i1 * jnp.int32(n2)
    w1 = q1_ref[bq] * jnp.float32(n1 - 1) - i1.astype(jnp.float32)
    w2 = q2_ref[bq] * jnp.float32(n2 - 1) - i2.astype(jnp.float32)
    w11 = w1 * w2
    w10 = w1 - w11
    w01 = w2 - w11
    w00 = (jnp.float32(1.0) - w1) - w01
    out_ref[...] = ((w00 * s00_ref[...] + w01 * s01_ref[...])
                    + (w10 * s10_ref[...] + w11 * s11_ref[...]))


def _tc_gather_lerp(q1t, q2t, table_flat, n1, n2, ne):
    bt = q1t.shape[0]
    t1 = q1t * jnp.float32(n1 - 1)
    t2 = q2t * jnp.float32(n2 - 1)
    i1 = jnp.minimum(t1.astype(jnp.int32), n1 - 2)
    i2 = jnp.minimum(t2.astype(jnp.int32), n2 - 2)
    r00 = i1 * n2 + i2

    table3 = table_flat.reshape(table_flat.shape[0], 1, ne)

    def corner(off):
        return pl.BlockSpec(
            (1, 1, ne),
            lambda bq, r00_ref, q1_ref, q2_ref: (r00_ref[bq] + off, 0, 0))

    grid_spec = pltpu.PrefetchScalarGridSpec(
        num_scalar_prefetch=3,
        grid=(bt,),
        in_specs=[corner(0), corner(1), corner(n2), corner(n2 + 1)],
        out_specs=pl.BlockSpec((1, 1, ne), lambda bq, *_: (bq, 0, 0)),
    )
    out3 = pl.pallas_call(
        functools.partial(_tc_lerp_body, n1, n2),
        grid_spec=grid_spec,
        out_shape=jax.ShapeDtypeStruct((bt, 1, ne), jnp.float32),
        compiler_params=pltpu.CompilerParams(
            dimension_semantics=("arbitrary",)),
    )(r00, q1t, q2t, table3, table3, table3, table3)
    return out3.reshape(bt, ne)


def kernel(param_values, spectra_table, grid1, grid2):
    n1, n2, ne = spectra_table.shape
    b = param_values.shape[0]
    nw = 32
    b_tc = 8192
    b_sc = b - b_tc
    nq = b_sc // nw

    table_flat = spectra_table.reshape(n1 * n2, ne)
    q1 = param_values[:, 0]
    q2 = param_values[:, 1]

    mesh = plsc.VectorSubcoreMesh(core_axis_name="c", subcore_axis_name="s")
    f = pl.kernel(
        functools.partial(_bilerp_kernel, n1, n2, ne, b_sc, nw),
        mesh=mesh,
        out_type=jax.ShapeDtypeStruct((b_sc, ne), jnp.float32),
        scratch_types=[
            pltpu.VMEM((nq,), jnp.float32),
            pltpu.VMEM((nq,), jnp.float32),
            pltpu.VMEM((4 * nq,), jnp.int32),
            pltpu.VMEM((8, ne), jnp.float32),
            pltpu.VMEM((8, ne), jnp.float32),
            pltpu.VMEM((2, ne), jnp.float32),
            pltpu.VMEM((2, ne), jnp.float32),
            pltpu.SemaphoreType.DMA,
            pltpu.SemaphoreType.DMA,
            pltpu.SemaphoreType.DMA,
            pltpu.SemaphoreType.DMA,
        ],
    )
    out_sc = f(q1[b_tc:], q2[b_tc:], table_flat)
    out_tc = _tc_gather_lerp(q1[:b_tc], q2[:b_tc], table_flat, n1, n2, ne)
    return jnp.concatenate([out_tc, out_sc], axis=0)


# speedup vs baseline: 7.3859x; 7.3859x over previous
import functools

import jax
import jax.numpy as jnp
from jax import lax
from jax.experimental import pallas as pl
from jax.experimental.pallas import tpu as pltpu
from jax.experimental.pallas import tpu_sc as plsc

L = 16


def _take(v, idx):
    return lax.gather(
        v, idx[:, None],
        dimension_numbers=lax.GatherDimensionNumbers(
            offset_dims=(), collapsed_slice_dims=(0,), start_index_map=(0,)),
        slice_sizes=(1,),
        mode=lax.GatherScatterMode.PROMISE_IN_BOUNDS)


def _bilerp_kernel(n1, n2, ne, b, nw, q1_hbm, q2_hbm, table_hbm, out_hbm,
                   q1_v, q2_v, idx_v, rbuf0, rbuf1, obuf0, obuf1,
                   sg0, sg1, so0, so1):
    nq = b // nw
    n_chunks = nq // 2
    wid = lax.axis_index("s") * 2 + lax.axis_index("c")
    qbase = wid * nq

    pltpu.sync_copy(q1_hbm.at[pl.ds(qbase, nq)], q1_v)
    pltpu.sync_copy(q2_hbm.at[pl.ds(qbase, nq)], q2_v)

    iota = lax.iota(jnp.int32, L)
    k = iota & 3
    pattern = (k & 1) + jnp.int32(n2) * (k >> 1)

    def cell(qvec, n):
        t = qvec * jnp.float32(n - 1)
        i = jnp.minimum(t.astype(jnp.int32), jnp.int32(n - 2))
        w = t - i.astype(jnp.float32)
        return i, w

    def group_vals(g):
        q1g = q1_v[pl.ds(g * L, L)]
        q2g = q2_v[pl.ds(g * L, L)]
        i1, w1 = cell(q1g, n1)
        i2, w2 = cell(q2g, n2)
        r00 = i1 * jnp.int32(n2) + i2
        return r00, w1, w2

    @plsc.parallel_loop(0, nq // 4, unroll=4)
    def build_idx(p):
        r00, _w1, _w2 = group_vals(p >> 2)
        sel = (4 * p) & (L - 1)
        idx_v[pl.ds(p * L, L)] = _take(r00, sel + (iota >> 2)) + pattern

    def fire_gather(c, rbuf, sem):
        pltpu.async_copy(table_hbm.at[idx_v.at[pl.ds(8 * c, 8)]], rbuf, sem)

    def wait_gather(rbuf, sem):
        pltpu.make_async_copy(
            table_hbm.at[idx_v.at[pl.ds(0, 8)]], rbuf, sem).wait()

    def compute(c, rbuf, obuf):
        r00, w1g, w2g = group_vals(c >> 3)
        lane = (2 * c) & (L - 1)
        one = jnp.float32(1.0)
        ws = []
        for jj in range(2):
            w1 = _take(w1g, jnp.full((L,), lane + jj, jnp.int32))
            w2 = _take(w2g, jnp.full((L,), lane + jj, jnp.int32))
            w11 = w1 * w2
            w10 = w1 - w11
            w01 = w2 - w11
            w00 = (one - w1) - w01
            ws.append((w00, w01, w10, w11))

        @plsc.parallel_loop(0, ne // L, unroll=8)
        def body(e):
            sl = pl.ds(e * L, L)
            for jj in range(2):
                w00, w01, w10, w11 = ws[jj]
                s00 = rbuf[4 * jj + 0, sl]
                s01 = rbuf[4 * jj + 1, sl]
                s10 = rbuf[4 * jj + 2, sl]
                s11 = rbuf[4 * jj + 3, sl]
                obuf[jj, sl] = \
                    (w00 * s00 + w01 * s01) + (w10 * s10 + w11 * s11)

    def fire_out(c, obuf, sem):
        pltpu.async_copy(obuf, out_hbm.at[pl.ds(qbase + 2 * c, 2)], sem)

    def wait_out(obuf, sem):
        pltpu.make_async_copy(obuf, out_hbm.at[pl.ds(0, 2)], sem).wait()

    fire_gather(0, rbuf0, sg0)
    fire_gather(1, rbuf1, sg1)

    def step(i, _):
        wait_gather(rbuf0, sg0)

        @pl.when(i > 0)
        def _():
            wait_out(obuf0, so0)

        compute(2 * i, rbuf0, obuf0)
        fire_out(2 * i, obuf0, so0)

        @pl.when(2 * i + 2 < n_chunks)
        def _():
            fire_gather(2 * i + 2, rbuf0, sg0)

        wait_gather(rbuf1, sg1)

        @pl.when(i > 0)
        def _():
            wait_out(obuf1, so1)

        compute(2 * i + 1, rbuf1, obuf1)
        fire_out(2 * i + 1, obuf1, so1)

        @pl.when(2 * i + 3 < n_chunks)
        def _():
            fire_gather(2 * i + 3, rbuf1, sg1)

        return 0

    lax.fori_loop(0, n_chunks // 2, step, 0, unroll=False)

    wait_out(obuf0, so0)
    wait_out(obuf1, so1)


def kernel(param_values, spectra_table, grid1, grid2):
    n1, n2, ne = spectra_table.shape
    b = param_values.shape[0]
    nw = 32
    nq = b // nw

    table_flat = spectra_table.reshape(n1 * n2, ne)
    q1 = param_values[:, 0]
    q2 = param_values[:, 1]

    mesh = plsc.VectorSubcoreMesh(core_axis_name="c", subcore_axis_name="s")
    f = pl.kernel(
        functools.partial(_bilerp_kernel, n1, n2, ne, b, nw),
        mesh=mesh,
        out_type=jax.ShapeDtypeStruct((b, ne), jnp.float32),
        scratch_types=[
            pltpu.VMEM((nq,), jnp.float32),
            pltpu.VMEM((nq,), jnp.float32),
            pltpu.VMEM((4 * nq,), jnp.int32),
            pltpu.VMEM((8, ne), jnp.float32),
            pltpu.VMEM((8, ne), jnp.float32),
            pltpu.VMEM((2, ne), jnp.float32),
            pltpu.VMEM((2, ne), jnp.float32),
            pltpu.SemaphoreType.DMA,
            pltpu.SemaphoreType.DMA,
            pltpu.SemaphoreType.DMA,
            pltpu.SemaphoreType.DMA,
        ],
    )
    return f(q1, q2, table_flat)
